# baseline (speedup 1.0000x reference)
import functools

import jax
import jax.numpy as jnp
from jax import lax
from jax.experimental import pallas as pl
from jax.experimental.pallas import tpu as pltpu
import jax.experimental.pallas.tpu_sc as plsc

_N = 10000
_NP = 10240
_E = 320000
_K = 80
_R = _E // _K
_RP = 4096
_C1 = _RP // 32
_C2 = _RP // 16
_B = 32
_NS = 16
_RT = _NP // _NS
_D = 128

_mesh = plsc.VectorSubcoreMesh(core_axis_name="c", subcore_axis_name="s")
_params = pltpu.CompilerParams(needs_layout_passes=False)
_f32 = jnp.float32
_i32 = jnp.int32


def _sp(v):
    return jnp.zeros((16,), _i32) + v


def _zero_my_slice(s, rows, zsh):
    def bz(i, _):
        for w in range(8):
            rows[i, pl.ds(w * 16, 16)] = jnp.zeros((16,), _f32)
        return None

    lax.fori_loop(0, _K, bz, None)
    for t in range(8):
        pltpu.sync_copy(rows, zsh.at[pl.ds(s * _RT + t * _K, _K)])


def _scale_scatter(j, ewv, rows, dstv, zsh):
    jv = _sp(j)

    @plsc.parallel_loop(0, _K, unroll=8, carry=_sp(0))
    def bi(i, iv):
        nb = plsc.load_gather(ewv, [jv, iv])
        for w in range(8):
            dsl = pl.ds(w * 16, 16)
            rows[i, dsl] = rows[i, dsl] * nb
        return iv + 1

    pltpu.sync_copy(rows, zsh.at[dstv.at[j]], add=True)


def _agg_batches(nbatch, ebase, table, src_hbm, dst_hbm, ew_hbm,
                 srcv, dstv, ewv, rows_a, rows_b, zsh, sem_a, sem_b,
                 dinvv=None):
    def bb(b, _):
        cb = ebase + b * _B
        pltpu.sync_copy(src_hbm.at[pl.ds(cb, _B)], srcv)
        pltpu.sync_copy(dst_hbm.at[pl.ds(cb, _B)], dstv)
        pltpu.sync_copy(ew_hbm.at[pl.ds(cb, _B)], ewv)

        if dinvv is not None:
            @plsc.parallel_loop(0, _B, unroll=2)
            def bn(j):
                for i in range(_K // 16):
                    dsl = pl.ds(i * 16, 16)
                    ns = plsc.load_gather(dinvv, [srcv[j, dsl]])
                    ewv[j, dsl] = ewv[j, dsl] * ns

        pltpu.async_copy(table.at[srcv.at[0]], rows_a, sem_a)

        def bj(j2, _):
            ja = 2 * j2
            jb = ja + 1
            pltpu.make_async_copy(table.at[srcv.at[ja]], rows_a, sem_a).wait()
            pltpu.async_copy(table.at[srcv.at[jb]], rows_b, sem_b)
            _scale_scatter(ja, ewv, rows_a, dstv, zsh)
            pltpu.make_async_copy(table.at[srcv.at[jb]], rows_b, sem_b).wait()

            @pl.when(j2 < _B // 2 - 1)
            def _():
                pltpu.async_copy(table.at[srcv.at[jb + 1]], rows_a, sem_a)

            _scale_scatter(jb, ewv, rows_b, dstv, zsh)
            return None

        lax.fori_loop(0, _B // 2, bj, None)
        return None

    lax.fori_loop(0, nbatch, bb, None)


@functools.partial(
    pl.kernel,
    out_type=jax.ShapeDtypeStruct((2, _NP), _f32),
    mesh=_mesh,
    compiler_params=_params,
    scratch_types=[
        pltpu.VMEM((_C1, _K), _i32),
        pltpu.VMEM((_C1, _K), _f32),
        pltpu.VMEM((_RT,), _f32),
        pltpu.VMEM_SHARED((_NP,), _f32),
    ],
)
def _deg(dst_hbm, ew_hbm, out_hbm, dstv, ewv, zv, dsh):
    c = lax.axis_index("c")
    s = lax.axis_index("s")

    def zz(i, _):
        zv[pl.ds(i * 16, 16)] = jnp.zeros((16,), _f32)
        return None

    lax.fori_loop(0, _RT // 16, zz, None)
    pltpu.sync_copy(zv, dsh.at[pl.ds(s * _RT, _RT)])
    plsc.subcore_barrier()

    base = (c * _NS + s) * _C1
    pltpu.sync_copy(dst_hbm.at[pl.ds(base, _C1)], dstv)
    pltpu.sync_copy(ew_hbm.at[pl.ds(base, _C1)], ewv)

    def bj(j, _):
        pltpu.sync_copy(ewv.at[j], dsh.at[dstv.at[j]], add=True)
        return None

    lax.fori_loop(0, _C1, bj, None)
    plsc.subcore_barrier()
    pltpu.sync_copy(dsh.at[pl.ds(s * _RT, _RT)], out_hbm.at[c, pl.ds(s * _RT, _RT)])


_sc_scratch = [
    pltpu.VMEM((_B, _K), _i32),
    pltpu.VMEM((_B, _K), _i32),
    pltpu.VMEM((_B, _K), _f32),
    pltpu.VMEM((_K, _D), _f32),
    pltpu.VMEM((_K, _D), _f32),
    pltpu.VMEM_SHARED((_NP, _D), _f32),
    pltpu.SemaphoreType.DMA,
    pltpu.SemaphoreType.DMA,
]


@functools.partial(
    pl.kernel,
    out_type=jax.ShapeDtypeStruct((2, _NP, _D), _f32),
    mesh=_mesh,
    compiler_params=_params,
    scratch_types=[pltpu.VMEM((_NP,), _f32)] + _sc_scratch,
)
def _agg1(x_hbm, src_hbm, dst_hbm, ew_hbm, dinv_hbm, out_hbm,
          dinvv, srcv, dstv, ewv, rows_a, rows_b, zsh, sem_a, sem_b):
    c = lax.axis_index("c")
    s = lax.axis_index("s")
    _zero_my_slice(s, rows_a, zsh)
    pltpu.sync_copy(dinv_hbm, dinvv)
    plsc.subcore_barrier()

    ebase = (c * _NS + s) * _C1
    _agg_batches(_C1 // _B, ebase, x_hbm, src_hbm, dst_hbm, ew_hbm,
                 srcv, dstv, ewv, rows_a, rows_b, zsh, sem_a, sem_b,
                 dinvv=dinvv)
    plsc.subcore_barrier()
    pltpu.sync_copy(zsh.at[pl.ds(s * _RT, _RT)],
                    out_hbm.at[c, pl.ds(s * _RT, _RT)])


@functools.partial(
    pl.kernel,
    out_type=jax.ShapeDtypeStruct((2, _NP, _D), _f32),
    mesh=_mesh,
    compiler_params=_params,
    scratch_types=_sc_scratch,
)
def _agg2(h0_hbm, h1_hbm, src_hbm, dst_hbm, ew_hbm, out_hbm,
          srcv, dstv, ewv, rows_a, rows_b, zsh, sem_a, sem_b):
    c = lax.axis_index("c")
    s = lax.axis_index("s")
    _zero_my_slice(s, rows_a, zsh)
    plsc.subcore_barrier()

    ebase = s * _C2

    @pl.when(c == 0)
    def _():
        _agg_batches(_C2 // _B, ebase, h0_hbm, src_hbm, dst_hbm, ew_hbm,
                     srcv, dstv, ewv, rows_a, rows_b, zsh, sem_a, sem_b)

    @pl.when(c == 1)
    def _():
        _agg_batches(_C2 // _B, ebase, h1_hbm, src_hbm, dst_hbm, ew_hbm,
                     srcv, dstv, ewv, rows_a, rows_b, zsh, sem_a, sem_b)

    plsc.subcore_barrier()
    pltpu.sync_copy(zsh.at[pl.ds(s * _RT, _RT)],
                    out_hbm.at[c, pl.ds(s * _RT, _RT)])


def _dinv_body(d_ref, o1_ref, o2_ref):
    d = d_ref[0] + d_ref[1] + 1.0
    o1_ref[...] = lax.rsqrt(d)
    o2_ref[...] = 1.0 / d


def _dinv(deg):
    shp = jax.ShapeDtypeStruct((_NP // 128, 128), _f32)
    o1, o2 = pl.pallas_call(
        _dinv_body,
        out_shape=(shp, shp),
    )(deg.reshape(2, _NP // 128, 128))
    return o1.reshape(_NP), o1.reshape(_NP, 1), o2.reshape(_NP, 1)


def _l1_body(z_ref, x_ref, d1_ref, d2_ref, w_ref, b_ref, o_ref, os_ref):
    z = d1_ref[...] * (z_ref[0] + z_ref[1]) + d2_ref[...] * x_ref[...]
    h = jnp.dot(z, w_ref[...], preferred_element_type=_f32) + b_ref[0]
    h = jnp.maximum(h, 0.0)
    o_ref[0] = h
    os_ref[0] = d1_ref[...] * h


def _l1(z1, xp, dinvc, dinv2c, W1, b1r):
    shp = jax.ShapeDtypeStruct((2, _NP, _D), _f32)
    return pl.pallas_call(
        _l1_body,
        grid=(_NP // 512, 2),
        in_specs=[
            pl.BlockSpec((2, 512, _D), lambda i, c: (0, i, 0)),
            pl.BlockSpec((512, _D), lambda i, c: (i, 0)),
            pl.BlockSpec((512, 1), lambda i, c: (i, 0)),
            pl.BlockSpec((512, 1), lambda i, c: (i, 0)),
            pl.BlockSpec((_D, _D), lambda i, c: (0, c)),
            pl.BlockSpec((1, 1, _D), lambda i, c: (c, 0, 0)),
        ],
        out_specs=(pl.BlockSpec((1, 512, _D), lambda i, c: (c, i, 0)),
                   pl.BlockSpec((1, 512, _D), lambda i, c: (c, i, 0))),
        out_shape=(shp, shp),
    )(z1, xp, dinvc, dinv2c, W1, b1r)


def _head_body(z_ref, h_ref, d1_ref, d2_ref, w2_ref, b2_ref, fw1_ref,
               fb1_ref, fw2_ref, fb2_ref, o_ref):
    d1 = d1_ref[...]
    d2 = d2_ref[...]
    a = jnp.concatenate(
        [d1 * z_ref[0] + d2 * h_ref[0], d1 * z_ref[1] + d2 * h_ref[1]],
        axis=1).astype(jnp.bfloat16)
    h = jnp.maximum(
        jnp.dot(a, w2_ref[...], preferred_element_type=_f32) + b2_ref[...], 0.0)
    h = jnp.maximum(
        jnp.dot(h.astype(jnp.bfloat16), fw1_ref[...],
                preferred_element_type=_f32) + fb1_ref[...],
        0.0)
    o = jnp.dot(h.astype(jnp.bfloat16), fw2_ref[...],
                preferred_element_type=_f32) + fb2_ref[...]
    m = jnp.max(o, axis=1, keepdims=True)
    e = jnp.exp(o - m)
    ssum = jnp.sum(e, axis=1, keepdims=True)
    o_ref[...] = o - m - jnp.log(ssum)


def _head(z2, h1h, dinvc, dinv2c, W2, b2r, FW1, Fb1r, FW2, Fb2r):
    nco = 40
    return pl.pallas_call(
        _head_body,
        grid=(_NP // 512,),
        in_specs=[
            pl.BlockSpec((2, 512, _D), lambda i: (0, i, 0)),
            pl.BlockSpec((2, 512, _D), lambda i: (0, i, 0)),
            pl.BlockSpec((512, 1), lambda i: (i, 0)),
            pl.BlockSpec((512, 1), lambda i: (i, 0)),
            pl.BlockSpec((256, 512), lambda i: (0, 0)),
            pl.BlockSpec((1, 512), lambda i: (0, 0)),
            pl.BlockSpec((512, 1024), lambda i: (0, 0)),
            pl.BlockSpec((1, 1024), lambda i: (0, 0)),
            pl.BlockSpec((1024, nco), lambda i: (0, 0)),
            pl.BlockSpec((1, nco), lambda i: (0, 0)),
        ],
        out_specs=pl.BlockSpec((512, nco), lambda i: (i, 0)),
        out_shape=jax.ShapeDtypeStruct((_NP, nco), _f32),
    )(z2, h1h, dinvc, dinv2c, W2, b2r, FW1, Fb1r, FW2, Fb2r)


def kernel(x, edge_index, edge_attr, W1, b1, W2, b2, FW1, Fb1, FW2, Fb2):
    pad_r = ((0, _RP - _R), (0, 0))
    pidx = (jnp.arange((_RP - _R) * _K, dtype=jnp.int32) % _NP).reshape(
        _RP - _R, _K)
    src2 = jnp.concatenate([edge_index[0].reshape(_R, _K), pidx])
    dst2 = jnp.concatenate([edge_index[1].reshape(_R, _K), pidx])
    ew2 = jnp.pad(edge_attr.reshape(_R, _K), pad_r)
    xp = jnp.pad(x, ((0, _NP - _N), (0, 0)))

    deg = _deg(dst2, ew2)
    dinvf, dinvc, dinv2c = _dinv(deg)
    z1 = _agg1(xp, src2, dst2, ew2, dinvf)
    h1h, h1s = _l1(z1, xp, dinvc, dinv2c, W1, b1.reshape(2, 1, _D))
    z2 = _agg2(h1s[0], h1s[1], src2, dst2, ew2)
    outp = _head(z2, h1h, dinvc, dinv2c, W2.astype(jnp.bfloat16),
                 b2.reshape(1, 512), FW1.astype(jnp.bfloat16),
                 Fb1.reshape(1, 1024), FW2.astype(jnp.bfloat16),
                 Fb2.reshape(1, 40))
    return outp[:_N]


# speedup vs baseline: 1.0327x; 1.0327x over previous
import functools

import jax
import jax.numpy as jnp
from jax import lax
from jax.experimental import pallas as pl
from jax.experimental.pallas import tpu as pltpu
import jax.experimental.pallas.tpu_sc as plsc

_N = 10000
_NP = 10240
_E = 320000
_K = 80
_R = _E // _K
_RP = 4096
_C1 = _RP // 32
_C2 = _RP // 16
_B = 32
_NS = 16
_RT = _NP // _NS
_D = 128

_mesh = plsc.VectorSubcoreMesh(core_axis_name="c", subcore_axis_name="s")
_params = pltpu.CompilerParams(needs_layout_passes=False)
_f32 = jnp.float32
_i32 = jnp.int32


def _sp(v):
    return jnp.zeros((16,), _i32) + v


def _zero_my_slice(s, rows, zsh):
    def bz(i, _):
        for w in range(8):
            rows[i, pl.ds(w * 16, 16)] = jnp.zeros((16,), _f32)
        return None

    lax.fori_loop(0, _K, bz, None)
    for t in range(8):
        pltpu.sync_copy(rows, zsh.at[pl.ds(s * _RT + t * _K, _K)])


def _scale(j, ewv, rows):
    jv = _sp(j)

    @plsc.parallel_loop(0, _K, unroll=8, carry=_sp(0))
    def bi(i, iv):
        nb = plsc.load_gather(ewv, [jv, iv])
        for w in range(8):
            dsl = pl.ds(w * 16, 16)
            rows[i, dsl] = rows[i, dsl] * nb
        return iv + 1


def _agg_batches(nbatch, ebase, table, src_hbm, dst_hbm, ew_hbm,
                 srcv, dstv, ewv, rows_a, rows_b, zsh, sem_a, sem_b,
                 sem_sa, sem_sb, dinvv=None):
    def bb(b, _):
        cb = ebase + b * _B
        pltpu.sync_copy(src_hbm.at[pl.ds(cb, _B)], srcv)
        pltpu.sync_copy(dst_hbm.at[pl.ds(cb, _B)], dstv)
        pltpu.sync_copy(ew_hbm.at[pl.ds(cb, _B)], ewv)

        if dinvv is not None:
            @plsc.parallel_loop(0, _B, unroll=2)
            def bn(j):
                for i in range(_K // 16):
                    dsl = pl.ds(i * 16, 16)
                    ns = plsc.load_gather(dinvv, [srcv[j, dsl]])
                    ewv[j, dsl] = ewv[j, dsl] * ns

        pltpu.async_copy(table.at[srcv.at[0]], rows_a, sem_a)
        pltpu.async_copy(table.at[srcv.at[1]], rows_b, sem_b)

        def bj(j2, _):
            ja = 2 * j2
            jb = ja + 1
            last = j2 == _B // 2 - 1
            pltpu.make_async_copy(table.at[srcv.at[ja]], rows_a, sem_a).wait()
            _scale(ja, ewv, rows_a)
            pltpu.async_copy(rows_a, zsh.at[dstv.at[ja]], sem_sa, add=True)
            pltpu.make_async_copy(table.at[srcv.at[jb]], rows_b, sem_b).wait()
            _scale(jb, ewv, rows_b)

            @pl.when(jnp.logical_not(last))
            def _():
                pltpu.make_async_copy(
                    rows_a, zsh.at[dstv.at[ja]], sem_sa).wait()
                pltpu.async_copy(table.at[srcv.at[ja + 2]], rows_a, sem_a)

            pltpu.async_copy(rows_b, zsh.at[dstv.at[jb]], sem_sb, add=True)

            @pl.when(jnp.logical_not(last))
            def _():
                pltpu.make_async_copy(
                    rows_b, zsh.at[dstv.at[jb]], sem_sb).wait()
                pltpu.async_copy(table.at[srcv.at[jb + 2]], rows_b, sem_b)

            return None

        lax.fori_loop(0, _B // 2, bj, None)
        pltpu.make_async_copy(rows_a, zsh.at[dstv.at[_B - 2]], sem_sa).wait()
        pltpu.make_async_copy(rows_b, zsh.at[dstv.at[_B - 1]], sem_sb).wait()
        return None

    lax.fori_loop(0, nbatch, bb, None)


@functools.partial(
    pl.kernel,
    out_type=jax.ShapeDtypeStruct((2, _NP), _f32),
    mesh=_mesh,
    compiler_params=_params,
    scratch_types=[
        pltpu.VMEM((_C1, _K), _i32),
        pltpu.VMEM((_C1, _K), _f32),
        pltpu.VMEM((_RT,), _f32),
        pltpu.VMEM_SHARED((_NP,), _f32),
    ],
)
def _deg(dst_hbm, ew_hbm, out_hbm, dstv, ewv, zv, dsh):
    c = lax.axis_index("c")
    s = lax.axis_index("s")

    def zz(i, _):
        zv[pl.ds(i * 16, 16)] = jnp.zeros((16,), _f32)
        return None

    lax.fori_loop(0, _RT // 16, zz, None)
    pltpu.sync_copy(zv, dsh.at[pl.ds(s * _RT, _RT)])
    plsc.subcore_barrier()

    base = (c * _NS + s) * _C1
    pltpu.sync_copy(dst_hbm.at[pl.ds(base, _C1)], dstv)
    pltpu.sync_copy(ew_hbm.at[pl.ds(base, _C1)], ewv)

    def bj(j, _):
        pltpu.sync_copy(ewv.at[j], dsh.at[dstv.at[j]], add=True)
        return None

    lax.fori_loop(0, _C1, bj, None)
    plsc.subcore_barrier()
    pltpu.sync_copy(dsh.at[pl.ds(s * _RT, _RT)], out_hbm.at[c, pl.ds(s * _RT, _RT)])


_sc_scratch = [
    pltpu.VMEM((_B, _K), _i32),
    pltpu.VMEM((_B, _K), _i32),
    pltpu.VMEM((_B, _K), _f32),
    pltpu.VMEM((_K, _D), _f32),
    pltpu.VMEM((_K, _D), _f32),
    pltpu.VMEM_SHARED((_NP, _D), _f32),
    pltpu.SemaphoreType.DMA,
    pltpu.SemaphoreType.DMA,
    pltpu.SemaphoreType.DMA,
    pltpu.SemaphoreType.DMA,
]


@functools.partial(
    pl.kernel,
    out_type=jax.ShapeDtypeStruct((2, _NP, _D), _f32),
    mesh=_mesh,
    compiler_params=_params,
    scratch_types=[pltpu.VMEM((_NP,), _f32)] + _sc_scratch,
)
def _agg1(x_hbm, src_hbm, dst_hbm, ew_hbm, dinv_hbm, out_hbm,
          dinvv, srcv, dstv, ewv, rows_a, rows_b, zsh, sem_a, sem_b,
          sem_sa, sem_sb):
    c = lax.axis_index("c")
    s = lax.axis_index("s")
    _zero_my_slice(s, rows_a, zsh)
    pltpu.sync_copy(dinv_hbm, dinvv)
    plsc.subcore_barrier()

    ebase = (c * _NS + s) * _C1
    _agg_batches(_C1 // _B, ebase, x_hbm, src_hbm, dst_hbm, ew_hbm,
                 srcv, dstv, ewv, rows_a, rows_b, zsh, sem_a, sem_b,
                 sem_sa, sem_sb, dinvv=dinvv)
    plsc.subcore_barrier()
    pltpu.sync_copy(zsh.at[pl.ds(s * _RT, _RT)],
                    out_hbm.at[c, pl.ds(s * _RT, _RT)])


@functools.partial(
    pl.kernel,
    out_type=jax.ShapeDtypeStruct((2, _NP, _D), _f32),
    mesh=_mesh,
    compiler_params=_params,
    scratch_types=_sc_scratch,
)
def _agg2(h0_hbm, h1_hbm, src_hbm, dst_hbm, ew_hbm, out_hbm,
          srcv, dstv, ewv, rows_a, rows_b, zsh, sem_a, sem_b,
          sem_sa, sem_sb):
    c = lax.axis_index("c")
    s = lax.axis_index("s")
    _zero_my_slice(s, rows_a, zsh)
    plsc.subcore_barrier()

    ebase = s * _C2

    @pl.when(c == 0)
    def _():
        _agg_batches(_C2 // _B, ebase, h0_hbm, src_hbm, dst_hbm, ew_hbm,
                     srcv, dstv, ewv, rows_a, rows_b, zsh, sem_a, sem_b,
                     sem_sa, sem_sb)

    @pl.when(c == 1)
    def _():
        _agg_batches(_C2 // _B, ebase, h1_hbm, src_hbm, dst_hbm, ew_hbm,
                     srcv, dstv, ewv, rows_a, rows_b, zsh, sem_a, sem_b,
                     sem_sa, sem_sb)

    plsc.subcore_barrier()
    pltpu.sync_copy(zsh.at[pl.ds(s * _RT, _RT)],
                    out_hbm.at[c, pl.ds(s * _RT, _RT)])


def _dinv_body(d_ref, o1_ref, o2_ref):
    d = d_ref[0] + d_ref[1] + 1.0
    o1_ref[...] = lax.rsqrt(d)
    o2_ref[...] = 1.0 / d


def _dinv(deg):
    shp = jax.ShapeDtypeStruct((_NP // 128, 128), _f32)
    o1, o2 = pl.pallas_call(
        _dinv_body,
        out_shape=(shp, shp),
    )(deg.reshape(2, _NP // 128, 128))
    return o1.reshape(_NP), o1.reshape(_NP, 1), o2.reshape(_NP, 1)


def _l1_body(z_ref, x_ref, d1_ref, d2_ref, w_ref, b_ref, o_ref, os_ref):
    z = d1_ref[...] * (z_ref[0] + z_ref[1]) + d2_ref[...] * x_ref[...]
    h = jnp.dot(z, w_ref[...], preferred_element_type=_f32) + b_ref[0]
    h = jnp.maximum(h, 0.0)
    o_ref[0] = h
    os_ref[0] = d1_ref[...] * h


def _l1(z1, xp, dinvc, dinv2c, W1, b1r):
    shp = jax.ShapeDtypeStruct((2, _NP, _D), _f32)
    return pl.pallas_call(
        _l1_body,
        grid=(_NP // 512, 2),
        in_specs=[
            pl.BlockSpec((2, 512, _D), lambda i, c: (0, i, 0)),
            pl.BlockSpec((512, _D), lambda i, c: (i, 0)),
            pl.BlockSpec((512, 1), lambda i, c: (i, 0)),
            pl.BlockSpec((512, 1), lambda i, c: (i, 0)),
            pl.BlockSpec((_D, _D), lambda i, c: (0, c)),
            pl.BlockSpec((1, 1, _D), lambda i, c: (c, 0, 0)),
        ],
        out_specs=(pl.BlockSpec((1, 512, _D), lambda i, c: (c, i, 0)),
                   pl.BlockSpec((1, 512, _D), lambda i, c: (c, i, 0))),
        out_shape=(shp, shp),
    )(z1, xp, dinvc, dinv2c, W1, b1r)


def _head_body(z_ref, h_ref, d1_ref, d2_ref, w2_ref, b2_ref, fw1_ref,
               fb1_ref, fw2_ref, fb2_ref, o_ref):
    d1 = d1_ref[...]
    d2 = d2_ref[...]
    a = jnp.concatenate(
        [d1 * z_ref[0] + d2 * h_ref[0], d1 * z_ref[1] + d2 * h_ref[1]],
        axis=1).astype(jnp.bfloat16)
    h = jnp.maximum(
        jnp.dot(a, w2_ref[...], preferred_element_type=_f32) + b2_ref[...], 0.0)
    h = jnp.maximum(
        jnp.dot(h.astype(jnp.bfloat16), fw1_ref[...],
                preferred_element_type=_f32) + fb1_ref[...],
        0.0)
    o = jnp.dot(h.astype(jnp.bfloat16), fw2_ref[...],
                preferred_element_type=_f32) + fb2_ref[...]
    m = jnp.max(o, axis=1, keepdims=True)
    e = jnp.exp(o - m)
    ssum = jnp.sum(e, axis=1, keepdims=True)
    o_ref[...] = o - m - jnp.log(ssum)


def _head(z2, h1h, dinvc, dinv2c, W2, b2r, FW1, Fb1r, FW2, Fb2r):
    nco = 40
    return pl.pallas_call(
        _head_body,
        grid=(_NP // 512,),
        in_specs=[
            pl.BlockSpec((2, 512, _D), lambda i: (0, i, 0)),
            pl.BlockSpec((2, 512, _D), lambda i: (0, i, 0)),
            pl.BlockSpec((512, 1), lambda i: (i, 0)),
            pl.BlockSpec((512, 1), lambda i: (i, 0)),
            pl.BlockSpec((256, 512), lambda i: (0, 0)),
            pl.BlockSpec((1, 512), lambda i: (0, 0)),
            pl.BlockSpec((512, 1024), lambda i: (0, 0)),
            pl.BlockSpec((1, 1024), lambda i: (0, 0)),
            pl.BlockSpec((1024, nco), lambda i: (0, 0)),
            pl.BlockSpec((1, nco), lambda i: (0, 0)),
        ],
        out_specs=pl.BlockSpec((512, nco), lambda i: (i, 0)),
        out_shape=jax.ShapeDtypeStruct((_NP, nco), _f32),
    )(z2, h1h, dinvc, dinv2c, W2, b2r, FW1, Fb1r, FW2, Fb2r)


def kernel(x, edge_index, edge_attr, W1, b1, W2, b2, FW1, Fb1, FW2, Fb2):
    pad_r = ((0, _RP - _R), (0, 0))
    pidx = (jnp.arange((_RP - _R) * _K, dtype=jnp.int32) % _NP).reshape(
        _RP - _R, _K)
    src2 = jnp.concatenate([edge_index[0].reshape(_R, _K), pidx])
    dst2 = jnp.concatenate([edge_index[1].reshape(_R, _K), pidx])
    ew2 = jnp.pad(edge_attr.reshape(_R, _K), pad_r)
    xp = jnp.pad(x, ((0, _NP - _N), (0, 0)))

    deg = _deg(dst2, ew2)
    dinvf, dinvc, dinv2c = _dinv(deg)
    z1 = _agg1(xp, src2, dst2, ew2, dinvf)
    h1h, h1s = _l1(z1, xp, dinvc, dinv2c, W1, b1.reshape(2, 1, _D))
    z2 = _agg2(h1s[0], h1s[1], src2, dst2, ew2)
    outp = _head(z2, h1h, dinvc, dinv2c, W2.astype(jnp.bfloat16),
                 b2.reshape(1, 512), FW1.astype(jnp.bfloat16),
                 Fb1.reshape(1, 1024), FW2.astype(jnp.bfloat16),
                 Fb2.reshape(1, 40))
    return outp[:_N]
